# baseline (speedup 1.0000x reference)
import functools

import jax
import jax.numpy as jnp
from jax import lax
from jax.experimental import pallas as pl
from jax.experimental.pallas import tpu as pltpu
from jax.experimental.pallas import tpu_sc as plsc

D = 128
CH = 128

_info = plsc.get_sparse_core_info()
NC = _info.num_cores
NS = _info.num_subcores
NW = NC * NS


def _make_gather(B: int):
    assert B % (NW * CH) == 0
    bpw = B // NW
    nchunk = bpw // CH
    assert nchunk % 2 == 0

    mesh = plsc.VectorSubcoreMesh(core_axis_name="c", subcore_axis_name="s")

    @functools.partial(
        pl.kernel,
        out_type=jax.ShapeDtypeStruct((B, D), jnp.float32),
        mesh=mesh,
        scratch_types=[
            pltpu.VMEM((nchunk, CH), jnp.int32),
            pltpu.VMEM((2, CH, D), jnp.float32),
            pltpu.SemaphoreType.DMA((2,)),
        ],
    )
    def gather_kernel(table_hbm, idx_hbm, out_hbm, idx_v, rows_v, sem):
        wid = lax.axis_index("s") * NC + lax.axis_index("c")
        base = wid * bpw
        pltpu.sync_copy(idx_hbm.at[wid], idx_v)

        def start_gather(c, b):
            pltpu.async_copy(table_hbm.at[idx_v.at[c]], rows_v.at[b], sem.at[b])

        def wait_gather(c, b):
            pltpu.make_async_copy(
                table_hbm.at[idx_v.at[c]], rows_v.at[b], sem.at[b]
            ).wait()

        start_gather(0, 0)

        def pair_body(p, carry):
            for b in range(2):
                c = 2 * p + b

                @pl.when(c + 1 < nchunk)
                def _():
                    start_gather(c + 1, 1 - b)

                wait_gather(c, b)
                pltpu.sync_copy(rows_v.at[b], out_hbm.at[pl.ds(base + c * CH, CH)])
            return carry

        lax.fori_loop(0, nchunk // 2, pair_body, 0)

    return gather_kernel


def kernel(token_ids, weight):
    shape = token_ids.shape
    B = token_ids.size
    idx = token_ids.reshape(NW, B // (NW * CH), CH).astype(jnp.int32)
    out = _make_gather(B)(weight, idx)
    return out.reshape(shape + (D,))


# speedup vs baseline: 1.7721x; 1.7721x over previous
import functools

import jax
import jax.numpy as jnp
from jax import lax
from jax.experimental import pallas as pl
from jax.experimental.pallas import tpu as pltpu
from jax.experimental.pallas import tpu_sc as plsc

D = 128
S = 50
SP = 56
RS = 8

_info = plsc.get_sparse_core_info()
NC = _info.num_cores
NS = _info.num_subcores
NW = NC * NS


def _make_gather(R: int):
    rpw = R // NW
    nslab = rpw // RS
    assert rpw % RS == 0 and nslab % 2 == 0

    mesh = plsc.VectorSubcoreMesh(core_axis_name="c", subcore_axis_name="s")

    @functools.partial(
        pl.kernel,
        out_type=jax.ShapeDtypeStruct((R, S, D), jnp.float32),
        mesh=mesh,
        scratch_types=[
            pltpu.VMEM((rpw, SP), jnp.int32),
            pltpu.VMEM((2, RS, S, D), jnp.float32),
            pltpu.SemaphoreType.DMA((2,)),
        ],
    )
    def gather_kernel(table_hbm, idx_hbm, out_hbm, idx_v, buf_v, sem):
        wid = lax.axis_index("s") * NC + lax.axis_index("c")
        rbase = wid * rpw
        pltpu.sync_copy(idx_hbm.at[wid], idx_v)

        def start_slab(s, b):
            for i in range(RS):
                pltpu.async_copy(
                    table_hbm.at[idx_v.at[s * RS + i, pl.ds(0, S)]],
                    buf_v.at[b, i],
                    sem.at[b],
                )

        def drain_slab(s, b):
            for i in range(RS):
                pltpu.make_async_copy(
                    table_hbm.at[idx_v.at[s * RS + i, pl.ds(0, S)]],
                    buf_v.at[b, i],
                    sem.at[b],
                ).wait()

        start_slab(0, 0)

        def pair_body(p, carry):
            for b in range(2):
                s = 2 * p + b

                @pl.when(s + 1 < nslab)
                def _():
                    start_slab(s + 1, 1 - b)

                drain_slab(s, b)
                pltpu.sync_copy(
                    buf_v.at[b], out_hbm.at[pl.ds(rbase + s * RS, RS)]
                )
            return carry

        lax.fori_loop(0, nslab // 2, pair_body, 0)

    return gather_kernel


def kernel(token_ids, weight):
    R, s = token_ids.shape
    idx = jnp.pad(token_ids.astype(jnp.int32), ((0, 0), (0, SP - s)))
    idx = idx.reshape(NW, R // NW, SP)
    return _make_gather(R)(weight, idx)


# speedup vs baseline: 3.1043x; 1.7518x over previous
import functools

import jax
import jax.numpy as jnp
from jax import lax
from jax.experimental import pallas as pl
from jax.experimental.pallas import tpu as pltpu
from jax.experimental.pallas import tpu_sc as plsc

D = 128
CH = 128

_info = plsc.get_sparse_core_info()
NC = _info.num_cores
NS = _info.num_subcores
NW = NC * NS


def _make_gather(B: int):
    assert B % (NW * CH) == 0
    bpw = B // NW
    nchunk = bpw // CH
    assert nchunk % 2 == 0

    mesh = plsc.VectorSubcoreMesh(core_axis_name="c", subcore_axis_name="s")

    @functools.partial(
        pl.kernel,
        out_type=jax.ShapeDtypeStruct((B, D), jnp.float32),
        mesh=mesh,
        scratch_types=[
            pltpu.VMEM((nchunk, CH), jnp.int32),
            pltpu.VMEM((2, CH, D), jnp.float32),
            pltpu.SemaphoreType.DMA((2,)),
        ],
    )
    def gather_kernel(table_hbm, idx_hbm, out_hbm, idx_v, rows_v, sem):
        wid = lax.axis_index("s") * NC + lax.axis_index("c")
        base = wid * bpw
        pltpu.sync_copy(idx_hbm.at[wid], idx_v)

        def start_gather(c, b):
            pltpu.async_copy(table_hbm.at[idx_v.at[c]], rows_v.at[b], sem.at[b])

        def wait_gather(c, b):
            pltpu.make_async_copy(
                table_hbm.at[idx_v.at[c]], rows_v.at[b], sem.at[b]
            ).wait()

        start_gather(0, 0)

        def pair_body(p, carry):
            for b in range(2):
                c = 2 * p + b

                @pl.when(c + 1 < nchunk)
                def _():
                    start_gather(c + 1, 1 - b)

                wait_gather(c, b)
                pltpu.sync_copy(rows_v.at[b], out_hbm.at[pl.ds(base + c * CH, CH)])
            return carry

        lax.fori_loop(0, nchunk // 2, pair_body, 0)

    return gather_kernel


def kernel(token_ids, weight):
    R, S = token_ids.shape
    B = token_ids.size
    idx = token_ids.T.astype(jnp.int32).reshape(NW, B // (NW * CH), CH)
    out = _make_gather(B)(weight, idx)
    return jnp.transpose(out.reshape(S, R, D), (1, 0, 2))


# speedup vs baseline: 3.1387x; 1.0111x over previous
import functools

import jax
import jax.numpy as jnp
from jax import lax
from jax.experimental import pallas as pl
from jax.experimental.pallas import tpu as pltpu
from jax.experimental.pallas import tpu_sc as plsc

D = 128
CH = 128

_info = plsc.get_sparse_core_info()
NC = _info.num_cores
NS = _info.num_subcores
NW = NC * NS


def _make_gather(B: int):
    assert B % (NW * CH) == 0
    bpw = B // NW
    nchunk = bpw // CH
    assert nchunk % 2 == 0

    mesh = plsc.VectorSubcoreMesh(core_axis_name="c", subcore_axis_name="s")

    @functools.partial(
        pl.kernel,
        out_type=jax.ShapeDtypeStruct((B, D), jnp.float32),
        mesh=mesh,
        scratch_types=[
            pltpu.VMEM((nchunk, CH), jnp.int32),
            pltpu.VMEM((2, CH, D), jnp.float32),
            pltpu.VMEM_SHARED((NS, 2, CH, D), jnp.float32),
            pltpu.SemaphoreType.DMA((2,)),
            pltpu.SemaphoreType.DMA((2,)),
        ],
    )
    def gather_kernel(table_hbm, idx_hbm, out_hbm, idx_v, rows_v, sh_v, gsem, ssem):
        cid = lax.axis_index("c")
        sid = lax.axis_index("s")
        wid = sid * NC + cid
        base = wid * bpw
        pltpu.sync_copy(idx_hbm.at[wid], idx_v)

        def start_gather(c, b):
            pltpu.async_copy(table_hbm.at[idx_v.at[c]], rows_v.at[b], gsem.at[b])

        def wait_gather(c, b):
            pltpu.make_async_copy(
                table_hbm.at[idx_v.at[c]], rows_v.at[b], gsem.at[b]
            ).wait()

        def store_desc(c, b):
            return pltpu.make_async_copy(
                sh_v.at[sid, b], out_hbm.at[pl.ds(base + c * CH, CH)], ssem.at[b]
            )

        start_gather(0, 0)

        def pair_body(p, carry):
            for b in range(2):
                c = 2 * p + b

                @pl.when(c + 1 < nchunk)
                def _():
                    start_gather(c + 1, 1 - b)

                wait_gather(c, b)

                @pl.when(c >= 2)
                def _():
                    store_desc(c - 2, b).wait()

                pltpu.sync_copy(rows_v.at[b], sh_v.at[sid, b])
                store_desc(c, b).start()
            return carry

        lax.fori_loop(0, nchunk // 2, pair_body, 0)

        for c in (nchunk - 2, nchunk - 1):
            store_desc(c, c % 2).wait()

    return gather_kernel


def kernel(token_ids, weight):
    R, S = token_ids.shape
    B = token_ids.size
    idx = token_ids.T.astype(jnp.int32).reshape(NW, B // (NW * CH), CH)
    out = _make_gather(B)(weight, idx)
    return jnp.transpose(out.reshape(S, R, D), (1, 0, 2))
